# baseline (speedup 1.0000x reference)
import functools

import jax
import jax.numpy as jnp
from jax import lax
from jax.experimental import pallas as pl
from jax.experimental.pallas import tpu as pltpu
from jax.experimental.pallas import tpu_sc as plsc

N = 10000
E = 320000
H = 128
G = 16
C = 10

NC = 2
NS = 16
NW = NC * NS
EPW = E // NW
CHUNK = 40
NCHUNKS = EPW // CHUNK
ROWS_PS = 624
TAIL = N - NS * ROWS_PS

_sc_mesh = plsc.VectorSubcoreMesh(core_axis_name="c", subcore_axis_name="s")


RB = 7


@functools.partial(
    pl.kernel,
    out_type=jax.ShapeDtypeStruct((NC, N, H), jnp.float32),
    mesh=_sc_mesh,
    scratch_types=[
        pltpu.VMEM((EPW,), jnp.int32),
        pltpu.VMEM((RB, CHUNK), jnp.int32),
        pltpu.VMEM((RB, CHUNK, H), jnp.float32),
        pltpu.VMEM_SHARED((N, H), jnp.float32),
        pltpu.SemaphoreType.DMA((RB,)),
        pltpu.SemaphoreType.DMA((RB,)),
        pltpu.SemaphoreType.DMA((RB,)),
        pltpu.SemaphoreType.DMA,
    ],
)
def _edge_agg(x_hbm, src_hbm, dst_hbm, init_hbm, out_hbm,
              src_v, dst_v, rows, agg_s, sem_i, sem_g, sem_s, sem0):
    cid = lax.axis_index("c")
    sid = lax.axis_index("s")
    wid = sid * NC + cid
    base = wid * EPW
    r0 = sid * ROWS_PS

    idx_cp1 = pltpu.async_copy(src_hbm.at[pl.ds(base, EPW)], src_v, sem0)

    @pl.when(cid == 0)
    def _():
        pltpu.async_copy(x_hbm.at[pl.ds(r0, ROWS_PS)],
                         agg_s.at[pl.ds(r0, ROWS_PS)], sem0).wait()

        @pl.when(sid == NS - 1)
        def _():
            pltpu.async_copy(x_hbm.at[pl.ds(NS * ROWS_PS, TAIL)],
                             agg_s.at[pl.ds(NS * ROWS_PS, TAIL)], sem0).wait()

    @pl.when(cid != 0)
    def _():
        pltpu.async_copy(init_hbm.at[pl.ds(r0, ROWS_PS)],
                         agg_s.at[pl.ds(r0, ROWS_PS)], sem0).wait()

        @pl.when(sid == NS - 1)
        def _():
            pltpu.async_copy(init_hbm.at[pl.ds(NS * ROWS_PS, TAIL)],
                             agg_s.at[pl.ds(NS * ROWS_PS, TAIL)], sem0).wait()

    idx_cp1.wait()
    plsc.subcore_barrier()

    def gather_desc(c, j):
        return pltpu.make_async_copy(
            x_hbm.at[src_v.at[pl.ds(c * CHUNK, CHUNK)]], rows.at[j],
            sem_g.at[j])

    def dst_desc(c, j):
        return pltpu.make_async_copy(
            dst_hbm.at[pl.ds(base + c * CHUNK, CHUNK)], dst_v.at[j],
            sem_i.at[j])

    def scatter_desc(j):
        return pltpu.make_async_copy(rows.at[j], agg_s.at[dst_v.at[j]],
                                     sem_s.at[j])

    DS = 6
    NR = (NCHUNKS + RB + RB - 1) // RB

    @pl.loop(0, NR)
    def _(r):
        for j in range(RB):
            i = r * RB + j

            @pl.when(jnp.logical_and(i >= RB, i < NCHUNKS + RB))
            def _():
                scatter_desc(j).wait()

            @pl.when(i < NCHUNKS)
            def _():
                gather_desc(i, j).start()
                dst_desc(i, j).start()

            js = (j - DS) % RB

            @pl.when(jnp.logical_and(i >= DS, i < NCHUNKS + DS))
            def _():
                gather_desc(0, js).wait()
                dst_desc(0, js).wait()
                pltpu.async_copy(rows.at[js], agg_s.at[dst_v.at[js]],
                                 sem_s.at[js], add=True)

    plsc.subcore_barrier()

    pltpu.sync_copy(agg_s.at[pl.ds(sid * ROWS_PS, ROWS_PS)],
                    out_hbm.at[cid, pl.ds(sid * ROWS_PS, ROWS_PS)])

    @pl.when(sid == NS - 1)
    def _():
        pltpu.sync_copy(agg_s.at[pl.ds(NS * ROWS_PS, TAIL)],
                        out_hbm.at[cid, pl.ds(NS * ROWS_PS, TAIL)])


def _elu(v):
    return jnp.where(v > 0, v, jnp.exp(jnp.minimum(v, 0.0)) - 1.0)


BLK = 1000


def _mlp1_body(p0_ref, p1_ref, w1_ref, b1_ref, w2_ref, b2_ref, o_ref):
    h = p0_ref[0] + p1_ref[0]
    h = _elu(jnp.dot(h, w1_ref[...], preferred_element_type=jnp.float32)
             + b1_ref[...])
    h = jnp.dot(h, w2_ref[...], preferred_element_type=jnp.float32) + b2_ref[...]
    o_ref[...] = _elu(h)


def _mlp1(parts, w1, b1, w2, b2):
    full = lambda a: pl.BlockSpec(a.shape, lambda i: (0,) * a.ndim)
    return pl.pallas_call(
        _mlp1_body,
        grid=(N // BLK,),
        in_specs=[
            pl.BlockSpec((1, BLK, H), lambda i: (0, i, 0)),
            pl.BlockSpec((1, BLK, H), lambda i: (1, i, 0)),
            full(w1), full(b1), full(w2), full(b2),
        ],
        out_specs=pl.BlockSpec((BLK, H), lambda i: (i, 0)),
        out_shape=jax.ShapeDtypeStruct((N, H), jnp.float32),
    )(parts, parts, w1, b1, w2, b2)


def _mlp2_body(p0_ref, p1_ref, batch_ref,
               w1_ref, b1_ref, w2_ref, b2_ref,
               m1_ref, mb1_ref, m2_ref, mb2_ref, m3_ref, mb3_ref,
               o_ref, g_acc):
    i = pl.program_id(0)

    @pl.when(i == 0)
    def _():
        g_acc[...] = jnp.zeros((G, H), jnp.float32)

    h = p0_ref[0] + p1_ref[0]
    h = _elu(jnp.dot(h, w1_ref[...], preferred_element_type=jnp.float32)
             + b1_ref[...])
    h = jnp.dot(h, w2_ref[...], preferred_element_type=jnp.float32) + b2_ref[...]
    h = _elu(h)

    oneh = (batch_ref[...] ==
            lax.broadcasted_iota(jnp.int32, (1, G), 1)).astype(jnp.float32)
    g_acc[...] += lax.dot_general(oneh, h, (((0,), (0,)), ((), ())),
                                  preferred_element_type=jnp.float32)

    @pl.when(i == pl.num_programs(0) - 1)
    def _():
        g = g_acc[...]
        g = _elu(jnp.dot(g, m1_ref[...], preferred_element_type=jnp.float32)
                 + mb1_ref[...])
        g = _elu(jnp.dot(g, m2_ref[...], preferred_element_type=jnp.float32)
                 + mb2_ref[...])
        o_ref[...] = (jnp.dot(g, m3_ref[...], preferred_element_type=jnp.float32)
                      + mb3_ref[...])


def _mlp2(parts, batch2d, w1, b1, w2, b2, m1, mb1, m2, mb2, m3, mb3):
    full = lambda a: pl.BlockSpec(a.shape, lambda i: (0,) * a.ndim)
    return pl.pallas_call(
        _mlp2_body,
        grid=(N // BLK,),
        in_specs=[
            pl.BlockSpec((1, BLK, H), lambda i: (0, i, 0)),
            pl.BlockSpec((1, BLK, H), lambda i: (1, i, 0)),
            pl.BlockSpec((BLK, 1), lambda i: (i, 0)),
            full(w1), full(b1), full(w2), full(b2),
            full(m1), full(mb1), full(m2), full(mb2), full(m3), full(mb3),
        ],
        out_specs=pl.BlockSpec((G, C), lambda i: (0, 0)),
        out_shape=jax.ShapeDtypeStruct((G, C), jnp.float32),
        scratch_shapes=[pltpu.VMEM((G, H), jnp.float32)],
    )(parts, parts, batch2d, w1, b1, w2, b2,
      m1, mb1, m2, mb2, m3, mb3)


def kernel(x, edge_index, batch, pre_W1, pre_b1, pre_W2, pre_b2,
           post_W1, post_b1, post_W2, post_b2,
           mlp_W1, mlp_b1, mlp_W2, mlp_b2, mlp_W3, mlp_b3):
    src = edge_index[0]
    dst = edge_index[1]
    batch2d = batch[:, None]
    zeros = jnp.zeros((N, H), jnp.float32)

    parts1 = _edge_agg(x, src, dst, zeros)
    h1 = _mlp1(parts1, pre_W1, pre_b1[None, :], pre_W2, pre_b2[None, :])
    parts2 = _edge_agg(h1, src, dst, zeros)
    out = _mlp2(parts2, batch2d,
                post_W1, post_b1[None, :], post_W2, post_b2[None, :],
                mlp_W1, mlp_b1[None, :], mlp_W2, mlp_b2[None, :],
                mlp_W3, mlp_b3[None, :])
    return out


# speedup vs baseline: 1.0023x; 1.0023x over previous
import functools

import jax
import jax.numpy as jnp
from jax import lax
from jax.experimental import pallas as pl
from jax.experimental.pallas import tpu as pltpu
from jax.experimental.pallas import tpu_sc as plsc

N = 10000
E = 320000
H = 128
G = 16
C = 10

NC = 2
NS = 16
NW = NC * NS
EPW = E // NW
CHUNK = 40
NCHUNKS = EPW // CHUNK
ROWS_PS = 624
TAIL = N - NS * ROWS_PS

_sc_mesh = plsc.VectorSubcoreMesh(core_axis_name="c", subcore_axis_name="s")


RB = 7


@functools.partial(
    pl.kernel,
    out_type=jax.ShapeDtypeStruct((NC, N, H), jnp.float32),
    mesh=_sc_mesh,
    scratch_types=[
        pltpu.VMEM((EPW,), jnp.int32),
        pltpu.VMEM((RB, CHUNK), jnp.int32),
        pltpu.VMEM((RB, CHUNK, H), jnp.float32),
        pltpu.VMEM_SHARED((N, H), jnp.float32),
        pltpu.SemaphoreType.DMA((RB,)),
        pltpu.SemaphoreType.DMA((RB,)),
        pltpu.SemaphoreType.DMA((RB,)),
        pltpu.SemaphoreType.DMA,
    ],
)
def _edge_agg(x_hbm, src_hbm, dst_hbm, init_hbm, out_hbm,
              src_v, dst_v, rows, agg_s, sem_i, sem_g, sem_s, sem0):
    cid = lax.axis_index("c")
    sid = lax.axis_index("s")
    wid = sid * NC + cid
    base = wid * EPW
    r0 = sid * ROWS_PS

    idx_cp1 = pltpu.async_copy(src_hbm.at[pl.ds(base, EPW)], src_v, sem0)

    @pl.when(cid == 0)
    def _():
        pltpu.async_copy(x_hbm.at[pl.ds(r0, ROWS_PS)],
                         agg_s.at[pl.ds(r0, ROWS_PS)], sem0).wait()

        @pl.when(sid == NS - 1)
        def _():
            pltpu.async_copy(x_hbm.at[pl.ds(NS * ROWS_PS, TAIL)],
                             agg_s.at[pl.ds(NS * ROWS_PS, TAIL)], sem0).wait()

    @pl.when(cid != 0)
    def _():
        pltpu.async_copy(init_hbm.at[pl.ds(r0, ROWS_PS)],
                         agg_s.at[pl.ds(r0, ROWS_PS)], sem0).wait()

        @pl.when(sid == NS - 1)
        def _():
            pltpu.async_copy(init_hbm.at[pl.ds(NS * ROWS_PS, TAIL)],
                             agg_s.at[pl.ds(NS * ROWS_PS, TAIL)], sem0).wait()

    idx_cp1.wait()
    plsc.subcore_barrier()

    def gather_desc(c, j):
        return pltpu.make_async_copy(
            x_hbm.at[src_v.at[pl.ds(c * CHUNK, CHUNK)]], rows.at[j],
            sem_g.at[j])

    def dst_desc(c, j):
        return pltpu.make_async_copy(
            dst_hbm.at[pl.ds(base + c * CHUNK, CHUNK)], dst_v.at[j],
            sem_i.at[j])

    def scatter_desc(j):
        return pltpu.make_async_copy(rows.at[j], agg_s.at[dst_v.at[j]],
                                     sem_s.at[j])

    DS = 5
    NR = (NCHUNKS + RB + RB - 1) // RB

    @pl.loop(0, NR)
    def _(r):
        for j in range(RB):
            i = r * RB + j

            @pl.when(jnp.logical_and(i >= RB, i < NCHUNKS + RB))
            def _():
                scatter_desc(j).wait()

            @pl.when(i < NCHUNKS)
            def _():
                gather_desc(i, j).start()
                dst_desc(i, j).start()

            js = (j - DS) % RB

            @pl.when(jnp.logical_and(i >= DS, i < NCHUNKS + DS))
            def _():
                gather_desc(0, js).wait()
                dst_desc(0, js).wait()
                pltpu.async_copy(rows.at[js], agg_s.at[dst_v.at[js]],
                                 sem_s.at[js], add=True)

    plsc.subcore_barrier()

    pltpu.sync_copy(agg_s.at[pl.ds(sid * ROWS_PS, ROWS_PS)],
                    out_hbm.at[cid, pl.ds(sid * ROWS_PS, ROWS_PS)])

    @pl.when(sid == NS - 1)
    def _():
        pltpu.sync_copy(agg_s.at[pl.ds(NS * ROWS_PS, TAIL)],
                        out_hbm.at[cid, pl.ds(NS * ROWS_PS, TAIL)])


def _elu(v):
    return jnp.where(v > 0, v, jnp.exp(jnp.minimum(v, 0.0)) - 1.0)


BLK = 1000


def _mlp1_body(p0_ref, p1_ref, w1_ref, b1_ref, w2_ref, b2_ref, o_ref):
    h = p0_ref[0] + p1_ref[0]
    h = _elu(jnp.dot(h, w1_ref[...], preferred_element_type=jnp.float32)
             + b1_ref[...])
    h = jnp.dot(h, w2_ref[...], preferred_element_type=jnp.float32) + b2_ref[...]
    o_ref[...] = _elu(h)


def _mlp1(parts, w1, b1, w2, b2):
    full = lambda a: pl.BlockSpec(a.shape, lambda i: (0,) * a.ndim)
    return pl.pallas_call(
        _mlp1_body,
        grid=(N // BLK,),
        in_specs=[
            pl.BlockSpec((1, BLK, H), lambda i: (0, i, 0)),
            pl.BlockSpec((1, BLK, H), lambda i: (1, i, 0)),
            full(w1), full(b1), full(w2), full(b2),
        ],
        out_specs=pl.BlockSpec((BLK, H), lambda i: (i, 0)),
        out_shape=jax.ShapeDtypeStruct((N, H), jnp.float32),
    )(parts, parts, w1, b1, w2, b2)


def _mlp2_body(p0_ref, p1_ref, batch_ref,
               w1_ref, b1_ref, w2_ref, b2_ref,
               m1_ref, mb1_ref, m2_ref, mb2_ref, m3_ref, mb3_ref,
               o_ref, g_acc):
    i = pl.program_id(0)

    @pl.when(i == 0)
    def _():
        g_acc[...] = jnp.zeros((G, H), jnp.float32)

    h = p0_ref[0] + p1_ref[0]
    h = _elu(jnp.dot(h, w1_ref[...], preferred_element_type=jnp.float32)
             + b1_ref[...])
    h = jnp.dot(h, w2_ref[...], preferred_element_type=jnp.float32) + b2_ref[...]
    h = _elu(h)

    oneh = (batch_ref[...] ==
            lax.broadcasted_iota(jnp.int32, (1, G), 1)).astype(jnp.float32)
    g_acc[...] += lax.dot_general(oneh, h, (((0,), (0,)), ((), ())),
                                  preferred_element_type=jnp.float32)

    @pl.when(i == pl.num_programs(0) - 1)
    def _():
        g = g_acc[...]
        g = _elu(jnp.dot(g, m1_ref[...], preferred_element_type=jnp.float32)
                 + mb1_ref[...])
        g = _elu(jnp.dot(g, m2_ref[...], preferred_element_type=jnp.float32)
                 + mb2_ref[...])
        o_ref[...] = (jnp.dot(g, m3_ref[...], preferred_element_type=jnp.float32)
                      + mb3_ref[...])


def _mlp2(parts, batch2d, w1, b1, w2, b2, m1, mb1, m2, mb2, m3, mb3):
    full = lambda a: pl.BlockSpec(a.shape, lambda i: (0,) * a.ndim)
    return pl.pallas_call(
        _mlp2_body,
        grid=(N // BLK,),
        in_specs=[
            pl.BlockSpec((1, BLK, H), lambda i: (0, i, 0)),
            pl.BlockSpec((1, BLK, H), lambda i: (1, i, 0)),
            pl.BlockSpec((BLK, 1), lambda i: (i, 0)),
            full(w1), full(b1), full(w2), full(b2),
            full(m1), full(mb1), full(m2), full(mb2), full(m3), full(mb3),
        ],
        out_specs=pl.BlockSpec((G, C), lambda i: (0, 0)),
        out_shape=jax.ShapeDtypeStruct((G, C), jnp.float32),
        scratch_shapes=[pltpu.VMEM((G, H), jnp.float32)],
    )(parts, parts, batch2d, w1, b1, w2, b2,
      m1, mb1, m2, mb2, m3, mb3)


def kernel(x, edge_index, batch, pre_W1, pre_b1, pre_W2, pre_b2,
           post_W1, post_b1, post_W2, post_b2,
           mlp_W1, mlp_b1, mlp_W2, mlp_b2, mlp_W3, mlp_b3):
    src = edge_index[0]
    dst = edge_index[1]
    batch2d = batch[:, None]
    zeros = jnp.zeros((N, H), jnp.float32)

    parts1 = _edge_agg(x, src, dst, zeros)
    h1 = _mlp1(parts1, pre_W1, pre_b1[None, :], pre_W2, pre_b2[None, :])
    parts2 = _edge_agg(h1, src, dst, zeros)
    out = _mlp2(parts2, batch2d,
                post_W1, post_b1[None, :], post_W2, post_b2[None, :],
                mlp_W1, mlp_b1[None, :], mlp_W2, mlp_b2[None, :],
                mlp_W3, mlp_b3[None, :])
    return out


# speedup vs baseline: 1.0244x; 1.0221x over previous
import functools

import jax
import jax.numpy as jnp
from jax import lax
from jax.experimental import pallas as pl
from jax.experimental.pallas import tpu as pltpu
from jax.experimental.pallas import tpu_sc as plsc

N = 10000
E = 320000
H = 128
G = 16
C = 10

NC = 2
NS = 16
NW = NC * NS
EPW = E // NW
CHUNK = 40
NCHUNKS = EPW // CHUNK
ROWS_PS = 624
TAIL = N - NS * ROWS_PS

_sc_mesh = plsc.VectorSubcoreMesh(core_axis_name="c", subcore_axis_name="s")


RB = 7


@functools.partial(
    pl.kernel,
    out_type=jax.ShapeDtypeStruct((NC, N, H), jnp.float32),
    mesh=_sc_mesh,
    scratch_types=[
        pltpu.VMEM((EPW,), jnp.int32),
        pltpu.VMEM((RB, CHUNK), jnp.int32),
        pltpu.VMEM((RB, CHUNK, H), jnp.float32),
        pltpu.VMEM_SHARED((N, H), jnp.float32),
        pltpu.SemaphoreType.DMA((RB,)),
        pltpu.SemaphoreType.DMA((RB,)),
        pltpu.SemaphoreType.DMA((RB,)),
        pltpu.SemaphoreType.DMA,
    ],
)
def _edge_agg(x_hbm, src_hbm, dst_hbm, init_hbm, out_hbm,
              src_v, dst_v, rows, agg_s, sem_i, sem_g, sem_s, sem0):
    cid = lax.axis_index("c")
    sid = lax.axis_index("s")
    wid = sid * NC + cid
    base = wid * EPW
    r0 = sid * ROWS_PS

    idx_cp1 = pltpu.async_copy(src_hbm.at[pl.ds(base, EPW)], src_v, sem0)

    @pl.when(cid == 0)
    def _():
        pltpu.async_copy(x_hbm.at[pl.ds(r0, ROWS_PS)],
                         agg_s.at[pl.ds(r0, ROWS_PS)], sem0).wait()

        @pl.when(sid == NS - 1)
        def _():
            pltpu.async_copy(x_hbm.at[pl.ds(NS * ROWS_PS, TAIL)],
                             agg_s.at[pl.ds(NS * ROWS_PS, TAIL)], sem0).wait()

    @pl.when(cid != 0)
    def _():
        pltpu.async_copy(init_hbm.at[pl.ds(r0, ROWS_PS)],
                         agg_s.at[pl.ds(r0, ROWS_PS)], sem0).wait()

        @pl.when(sid == NS - 1)
        def _():
            pltpu.async_copy(init_hbm.at[pl.ds(NS * ROWS_PS, TAIL)],
                             agg_s.at[pl.ds(NS * ROWS_PS, TAIL)], sem0).wait()

    idx_cp1.wait()
    plsc.subcore_barrier()

    def gather_desc(c, j):
        return pltpu.make_async_copy(
            x_hbm.at[src_v.at[pl.ds(c * CHUNK, CHUNK)]], rows.at[j],
            sem_g.at[j])

    def dst_desc(c, j):
        return pltpu.make_async_copy(
            dst_hbm.at[pl.ds(base + c * CHUNK, CHUNK)], dst_v.at[j],
            sem_i.at[j])

    def scatter_desc(j):
        return pltpu.make_async_copy(rows.at[j], agg_s.at[dst_v.at[j]],
                                     sem_s.at[j])

    DS = 5
    NR = (NCHUNKS + RB + RB - 1) // RB

    @pl.loop(0, NR)
    def _(r):
        for j in range(RB):
            i = r * RB + j

            @pl.when(jnp.logical_and(i >= RB, i < NCHUNKS + RB))
            def _():
                scatter_desc(j).wait()

            @pl.when(i < NCHUNKS)
            def _():
                gather_desc(i, j).start()
                dst_desc(i, j).start()

            js = (j - DS) % RB

            @pl.when(jnp.logical_and(i >= DS, i < NCHUNKS + DS))
            def _():
                gather_desc(0, js).wait()
                dst_desc(0, js).wait()
                pltpu.async_copy(rows.at[js], agg_s.at[dst_v.at[js]],
                                 sem_s.at[js], add=True)

    plsc.subcore_barrier()

    pltpu.sync_copy(agg_s.at[pl.ds(sid * ROWS_PS, ROWS_PS)],
                    out_hbm.at[cid, pl.ds(sid * ROWS_PS, ROWS_PS)])

    @pl.when(sid == NS - 1)
    def _():
        pltpu.sync_copy(agg_s.at[pl.ds(NS * ROWS_PS, TAIL)],
                        out_hbm.at[cid, pl.ds(NS * ROWS_PS, TAIL)])


def _elu(v):
    return jnp.where(v > 0, v, jnp.exp(jnp.minimum(v, 0.0)) - 1.0)


BLK = 2000


def _mlp1_body(p0_ref, p1_ref, w1_ref, b1_ref, w2_ref, b2_ref, o_ref):
    h = p0_ref[0] + p1_ref[0]
    h = _elu(jnp.dot(h, w1_ref[...], preferred_element_type=jnp.float32)
             + b1_ref[...])
    h = jnp.dot(h, w2_ref[...], preferred_element_type=jnp.float32) + b2_ref[...]
    o_ref[...] = _elu(h)


def _mlp1(parts, w1, b1, w2, b2):
    full = lambda a: pl.BlockSpec(a.shape, lambda i: (0,) * a.ndim)
    return pl.pallas_call(
        _mlp1_body,
        grid=(N // BLK,),
        in_specs=[
            pl.BlockSpec((1, BLK, H), lambda i: (0, i, 0)),
            pl.BlockSpec((1, BLK, H), lambda i: (1, i, 0)),
            full(w1), full(b1), full(w2), full(b2),
        ],
        out_specs=pl.BlockSpec((BLK, H), lambda i: (i, 0)),
        out_shape=jax.ShapeDtypeStruct((N, H), jnp.float32),
    )(parts, parts, w1, b1, w2, b2)


def _mlp2_body(p0_ref, p1_ref, batch_ref,
               w1_ref, b1_ref, w2_ref, b2_ref,
               m1_ref, mb1_ref, m2_ref, mb2_ref, m3_ref, mb3_ref,
               o_ref, g_acc):
    i = pl.program_id(0)

    @pl.when(i == 0)
    def _():
        g_acc[...] = jnp.zeros((G, H), jnp.float32)

    h = p0_ref[0] + p1_ref[0]
    h = _elu(jnp.dot(h, w1_ref[...], preferred_element_type=jnp.float32)
             + b1_ref[...])
    h = jnp.dot(h, w2_ref[...], preferred_element_type=jnp.float32) + b2_ref[...]
    h = _elu(h)

    oneh = (batch_ref[...] ==
            lax.broadcasted_iota(jnp.int32, (1, G), 1)).astype(jnp.float32)
    g_acc[...] += lax.dot_general(oneh, h, (((0,), (0,)), ((), ())),
                                  preferred_element_type=jnp.float32)

    @pl.when(i == pl.num_programs(0) - 1)
    def _():
        g = g_acc[...]
        g = _elu(jnp.dot(g, m1_ref[...], preferred_element_type=jnp.float32)
                 + mb1_ref[...])
        g = _elu(jnp.dot(g, m2_ref[...], preferred_element_type=jnp.float32)
                 + mb2_ref[...])
        o_ref[...] = (jnp.dot(g, m3_ref[...], preferred_element_type=jnp.float32)
                      + mb3_ref[...])


def _mlp2(parts, batch2d, w1, b1, w2, b2, m1, mb1, m2, mb2, m3, mb3):
    full = lambda a: pl.BlockSpec(a.shape, lambda i: (0,) * a.ndim)
    return pl.pallas_call(
        _mlp2_body,
        grid=(N // BLK,),
        in_specs=[
            pl.BlockSpec((1, BLK, H), lambda i: (0, i, 0)),
            pl.BlockSpec((1, BLK, H), lambda i: (1, i, 0)),
            pl.BlockSpec((BLK, 1), lambda i: (i, 0)),
            full(w1), full(b1), full(w2), full(b2),
            full(m1), full(mb1), full(m2), full(mb2), full(m3), full(mb3),
        ],
        out_specs=pl.BlockSpec((G, C), lambda i: (0, 0)),
        out_shape=jax.ShapeDtypeStruct((G, C), jnp.float32),
        scratch_shapes=[pltpu.VMEM((G, H), jnp.float32)],
    )(parts, parts, batch2d, w1, b1, w2, b2,
      m1, mb1, m2, mb2, m3, mb3)


def kernel(x, edge_index, batch, pre_W1, pre_b1, pre_W2, pre_b2,
           post_W1, post_b1, post_W2, post_b2,
           mlp_W1, mlp_b1, mlp_W2, mlp_b2, mlp_W3, mlp_b3):
    src = edge_index[0]
    dst = edge_index[1]
    batch2d = batch[:, None]
    zeros = jnp.zeros((N, H), jnp.float32)

    parts1 = _edge_agg(x, src, dst, zeros)
    h1 = _mlp1(parts1, pre_W1, pre_b1[None, :], pre_W2, pre_b2[None, :])
    parts2 = _edge_agg(h1, src, dst, zeros)
    out = _mlp2(parts2, batch2d,
                post_W1, post_b1[None, :], post_W2, post_b2[None, :],
                mlp_W1, mlp_b1[None, :], mlp_W2, mlp_b2[None, :],
                mlp_W3, mlp_b3[None, :])
    return out


# speedup vs baseline: 1.0410x; 1.0161x over previous
import functools

import jax
import jax.numpy as jnp
from jax import lax
from jax.experimental import pallas as pl
from jax.experimental.pallas import tpu as pltpu
from jax.experimental.pallas import tpu_sc as plsc

N = 10000
E = 320000
H = 128
G = 16
C = 10

NC = 2
NS = 16
NW = NC * NS
EPW = E // NW
CHUNK = 40
NCHUNKS = EPW // CHUNK
ROWS_PS = 624
TAIL = N - NS * ROWS_PS

_sc_mesh = plsc.VectorSubcoreMesh(core_axis_name="c", subcore_axis_name="s")


RB = 7


@functools.partial(
    pl.kernel,
    out_type=jax.ShapeDtypeStruct((NC, N, H), jnp.float32),
    mesh=_sc_mesh,
    scratch_types=[
        pltpu.VMEM((EPW,), jnp.int32),
        pltpu.VMEM((RB, CHUNK), jnp.int32),
        pltpu.VMEM((RB, CHUNK, H), jnp.float32),
        pltpu.VMEM_SHARED((N, H), jnp.float32),
        pltpu.SemaphoreType.DMA((RB,)),
        pltpu.SemaphoreType.DMA((RB,)),
        pltpu.SemaphoreType.DMA((RB,)),
        pltpu.SemaphoreType.DMA,
    ],
)
def _edge_agg(x_hbm, src_hbm, dst_hbm, init_hbm, out_hbm,
              src_v, dst_v, rows, agg_s, sem_i, sem_g, sem_s, sem0):
    cid = lax.axis_index("c")
    sid = lax.axis_index("s")
    wid = sid * NC + cid
    base = wid * EPW
    r0 = sid * ROWS_PS

    idx_cp1 = pltpu.async_copy(src_hbm.at[pl.ds(base, EPW)], src_v, sem0)

    @pl.when(cid == 0)
    def _():
        pltpu.async_copy(x_hbm.at[pl.ds(r0, ROWS_PS)],
                         agg_s.at[pl.ds(r0, ROWS_PS)], sem0).wait()

        @pl.when(sid == NS - 1)
        def _():
            pltpu.async_copy(x_hbm.at[pl.ds(NS * ROWS_PS, TAIL)],
                             agg_s.at[pl.ds(NS * ROWS_PS, TAIL)], sem0).wait()

    @pl.when(cid != 0)
    def _():
        pltpu.async_copy(init_hbm.at[pl.ds(r0, ROWS_PS)],
                         agg_s.at[pl.ds(r0, ROWS_PS)], sem0).wait()

        @pl.when(sid == NS - 1)
        def _():
            pltpu.async_copy(init_hbm.at[pl.ds(NS * ROWS_PS, TAIL)],
                             agg_s.at[pl.ds(NS * ROWS_PS, TAIL)], sem0).wait()

    idx_cp1.wait()
    plsc.subcore_barrier()

    def gather_desc(c, j):
        return pltpu.make_async_copy(
            x_hbm.at[src_v.at[pl.ds(c * CHUNK, CHUNK)]], rows.at[j],
            sem_g.at[j])

    def dst_desc(c, j):
        return pltpu.make_async_copy(
            dst_hbm.at[pl.ds(base + c * CHUNK, CHUNK)], dst_v.at[j],
            sem_i.at[j])

    def scatter_desc(j):
        return pltpu.make_async_copy(rows.at[j], agg_s.at[dst_v.at[j]],
                                     sem_s.at[j])

    DS = 5
    NR = (NCHUNKS + RB + RB - 1) // RB

    @pl.loop(0, NR)
    def _(r):
        for j in range(RB):
            i = r * RB + j

            @pl.when(jnp.logical_and(i >= RB, i < NCHUNKS + RB))
            def _():
                scatter_desc(j).wait()

            @pl.when(i < NCHUNKS)
            def _():
                gather_desc(i, j).start()
                dst_desc(i, j).start()

            js = (j - DS) % RB

            @pl.when(jnp.logical_and(i >= DS, i < NCHUNKS + DS))
            def _():
                gather_desc(0, js).wait()
                dst_desc(0, js).wait()
                pltpu.async_copy(rows.at[js], agg_s.at[dst_v.at[js]],
                                 sem_s.at[js], add=True)

    plsc.subcore_barrier()

    pltpu.sync_copy(agg_s.at[pl.ds(sid * ROWS_PS, ROWS_PS)],
                    out_hbm.at[cid, pl.ds(sid * ROWS_PS, ROWS_PS)])

    @pl.when(sid == NS - 1)
    def _():
        pltpu.sync_copy(agg_s.at[pl.ds(NS * ROWS_PS, TAIL)],
                        out_hbm.at[cid, pl.ds(NS * ROWS_PS, TAIL)])


def _elu(v):
    return jnp.where(v > 0, v, jnp.exp(jnp.minimum(v, 0.0)) - 1.0)


BLK = 5000


def _mlp1_body(p0_ref, p1_ref, w1_ref, b1_ref, w2_ref, b2_ref, o_ref):
    h = p0_ref[0] + p1_ref[0]
    h = _elu(jnp.dot(h, w1_ref[...], preferred_element_type=jnp.float32)
             + b1_ref[...])
    h = jnp.dot(h, w2_ref[...], preferred_element_type=jnp.float32) + b2_ref[...]
    o_ref[...] = _elu(h)


def _mlp1(parts, w1, b1, w2, b2):
    full = lambda a: pl.BlockSpec(a.shape, lambda i: (0,) * a.ndim)
    return pl.pallas_call(
        _mlp1_body,
        grid=(N // BLK,),
        in_specs=[
            pl.BlockSpec((1, BLK, H), lambda i: (0, i, 0)),
            pl.BlockSpec((1, BLK, H), lambda i: (1, i, 0)),
            full(w1), full(b1), full(w2), full(b2),
        ],
        out_specs=pl.BlockSpec((BLK, H), lambda i: (i, 0)),
        out_shape=jax.ShapeDtypeStruct((N, H), jnp.float32),
    )(parts, parts, w1, b1, w2, b2)


def _mlp2_body(p0_ref, p1_ref, batch_ref,
               w1_ref, b1_ref, w2_ref, b2_ref,
               m1_ref, mb1_ref, m2_ref, mb2_ref, m3_ref, mb3_ref,
               o_ref, g_acc):
    i = pl.program_id(0)

    @pl.when(i == 0)
    def _():
        g_acc[...] = jnp.zeros((G, H), jnp.float32)

    h = p0_ref[0] + p1_ref[0]
    h = _elu(jnp.dot(h, w1_ref[...], preferred_element_type=jnp.float32)
             + b1_ref[...])
    h = jnp.dot(h, w2_ref[...], preferred_element_type=jnp.float32) + b2_ref[...]
    h = _elu(h)

    oneh = (batch_ref[...] ==
            lax.broadcasted_iota(jnp.int32, (1, G), 1)).astype(jnp.float32)
    g_acc[...] += lax.dot_general(oneh, h, (((0,), (0,)), ((), ())),
                                  preferred_element_type=jnp.float32)

    @pl.when(i == pl.num_programs(0) - 1)
    def _():
        g = g_acc[...]
        g = _elu(jnp.dot(g, m1_ref[...], preferred_element_type=jnp.float32)
                 + mb1_ref[...])
        g = _elu(jnp.dot(g, m2_ref[...], preferred_element_type=jnp.float32)
                 + mb2_ref[...])
        o_ref[...] = (jnp.dot(g, m3_ref[...], preferred_element_type=jnp.float32)
                      + mb3_ref[...])


def _mlp2(parts, batch2d, w1, b1, w2, b2, m1, mb1, m2, mb2, m3, mb3):
    full = lambda a: pl.BlockSpec(a.shape, lambda i: (0,) * a.ndim)
    return pl.pallas_call(
        _mlp2_body,
        grid=(N // BLK,),
        in_specs=[
            pl.BlockSpec((1, BLK, H), lambda i: (0, i, 0)),
            pl.BlockSpec((1, BLK, H), lambda i: (1, i, 0)),
            pl.BlockSpec((BLK, 1), lambda i: (i, 0)),
            full(w1), full(b1), full(w2), full(b2),
            full(m1), full(mb1), full(m2), full(mb2), full(m3), full(mb3),
        ],
        out_specs=pl.BlockSpec((G, C), lambda i: (0, 0)),
        out_shape=jax.ShapeDtypeStruct((G, C), jnp.float32),
        scratch_shapes=[pltpu.VMEM((G, H), jnp.float32)],
    )(parts, parts, batch2d, w1, b1, w2, b2,
      m1, mb1, m2, mb2, m3, mb3)


def kernel(x, edge_index, batch, pre_W1, pre_b1, pre_W2, pre_b2,
           post_W1, post_b1, post_W2, post_b2,
           mlp_W1, mlp_b1, mlp_W2, mlp_b2, mlp_W3, mlp_b3):
    src = edge_index[0]
    dst = edge_index[1]
    batch2d = batch[:, None]
    zeros = jnp.zeros((N, H), jnp.float32)

    parts1 = _edge_agg(x, src, dst, zeros)
    h1 = _mlp1(parts1, pre_W1, pre_b1[None, :], pre_W2, pre_b2[None, :])
    parts2 = _edge_agg(h1, src, dst, zeros)
    out = _mlp2(parts2, batch2d,
                post_W1, post_b1[None, :], post_W2, post_b2[None, :],
                mlp_W1, mlp_b1[None, :], mlp_W2, mlp_b2[None, :],
                mlp_W3, mlp_b3[None, :])
    return out
